# baseline (speedup 1.0000x reference)
import functools

import jax
import jax.numpy as jnp
from jax import lax
from jax.experimental import pallas as pl
from jax.experimental.pallas import tpu as pltpu
from jax.experimental.pallas import tpu_sc as plsc

HID = 1024
VOCAB = 32000
MAXLEN = 32
MAXMEM = 512
SIMK = 3
CONTK = 2
WIN = 32
GAMMA = 2.0
B, S = 2, 2048
NTOK = B * S
VT = 128
NVT = VOCAB // VT

SC_NC, SC_NS = 2, 16
SC_NW = SC_NC * SC_NS
GROWS = 512
RPW = GROWS // SC_NW

_HI = jax.lax.Precision.HIGHEST


def _shift_lanes(x, k):
    n = x.shape[-1]
    z = jnp.zeros(x.shape[:-1] + (k,), x.dtype)
    return jnp.concatenate([z, x[..., : n - k]], axis=-1)


def _dcumsum(x):
    n = x.shape[-1]
    k = 1
    while k < n:
        x = x + _shift_lanes(x, k)
        k *= 2
    return x


def _lse_body(flat_ref, tgt_ref, w_ref, surp_ref, st_ref):
    i = pl.program_id(0)

    @pl.when(i == 0)
    def _init():
        st_ref[:, 0:1] = jnp.full((NTOK, 1), -1e30, jnp.float32)
        st_ref[:, 1:2] = jnp.zeros((NTOK, 1), jnp.float32)
        st_ref[:, 2:3] = jnp.zeros((NTOK, 1), jnp.float32)

    fl = flat_ref[...]
    wt = w_ref[...]
    lg = lax.dot_general(fl, wt, (((1,), (1,)), ((), ())),
                         precision=_HI, preferred_element_type=jnp.float32)
    col = i * VT + lax.broadcasted_iota(jnp.int32, (1, VT), 1)
    tgt = tgt_ref[...]
    st_ref[:, 2:3] = st_ref[:, 2:3] + jnp.sum(
        jnp.where(col == tgt, lg, 0.0), axis=1, keepdims=True)
    m_old = st_ref[:, 0:1]
    mt = jnp.max(lg, axis=1, keepdims=True)
    mn = jnp.maximum(m_old, mt)
    st_ref[:, 1:2] = st_ref[:, 1:2] * jnp.exp(m_old - mn) + jnp.sum(
        jnp.exp(lg - mn), axis=1, keepdims=True)
    st_ref[:, 0:1] = mn

    @pl.when(i == NVT - 1)
    def _fin():
        surp_ref[...] = st_ref[:, 0:1] + jnp.log(st_ref[:, 1:2]) - st_ref[:, 2:3]


def _surprise(flat, tgt2, W_clu):
    return pl.pallas_call(
        _lse_body,
        grid=(NVT,),
        in_specs=[
            pl.BlockSpec((NTOK, HID), lambda i: (0, 0)),
            pl.BlockSpec((NTOK, 1), lambda i: (0, 0)),
            pl.BlockSpec((VT, HID), lambda i: (i, 0)),
        ],
        out_specs=pl.BlockSpec((NTOK, 1), lambda i: (0, 0)),
        out_shape=jax.ShapeDtypeStruct((NTOK, 1), jnp.float32),
        scratch_shapes=[
            pltpu.VMEM((NTOK, 3), jnp.float32),
        ],
    )(flat, tgt2, W_clu)


def _plan_body(surp_ref, flat_ref, wsim_ref, bsim_ref, tok_ref, val_ref):
    s = surp_ref[...]
    cs = _dcumsum(s)
    cs2 = _dcumsum(s * s)
    smw = cs - _shift_lanes(cs, WIN)
    sm2w = cs2 - _shift_lanes(cs2, WIN)
    coli = lax.broadcasted_iota(jnp.int32, (1, S), 1)
    cnt = jnp.minimum(coli.astype(jnp.float32) + 1.0, float(WIN))
    mean = smw / cnt
    std = jnp.sqrt(jnp.clip(sm2w / cnt - mean * mean, 0.0, None))
    bnd = s > mean + GAMMA * std
    isnew = bnd | (coli == 0)
    inewi = isnew.astype(jnp.int32)
    R = _dcumsum(inewi)
    tot = R[:, S - 1:S]
    n_events = jnp.sum(tot)
    off = jnp.concatenate([jnp.zeros((1, 1), jnp.int32), tot[0:1]], axis=0)
    inv = R - 1 + off

    mm = lax.broadcasted_iota(jnp.int32, (1, MAXMEM), 1)
    written = mm < n_events
    qd = jnp.maximum(n_events - 1 - mm, 0) // MAXMEM
    e_m = mm + MAXMEM * qd

    countm = jnp.zeros((1, MAXMEM), jnp.int32)
    startm = jnp.zeros((1, MAXMEM), jnp.int32)
    ohs = []
    for b in range(B):
        invb = jnp.swapaxes(inv[b:b + 1, :], 0, 1)
        newb = jnp.swapaxes(inewi[b:b + 1, :], 0, 1)
        Mb = (invb == e_m) & written
        posb = b * S + lax.broadcasted_iota(jnp.int32, (S, 1), 0)
        ohs.append((Mb & (newb > 0)).astype(jnp.float32))
        countm = countm + jnp.sum(Mb.astype(jnp.int32), axis=0, keepdims=True)
        startm = startm + jnp.sum(jnp.where(Mb & (newb > 0), posb, 0),
                                  axis=0, keepdims=True)
    oh = jnp.concatenate(ohs, axis=0)

    fl = flat_ref[...]
    q0 = jnp.sum(fl[0:S], axis=0, keepdims=True) * (1.0 / S)
    q1 = jnp.sum(fl[S:NTOK], axis=0, keepdims=True) * (1.0 / S)
    qm = jnp.concatenate([q0, q1], axis=0)
    qs = lax.dot_general(qm, wsim_ref[...], (((1,), (1,)), ((), ())),
                         precision=_HI, preferred_element_type=jnp.float32)
    qs = qs + bsim_ref[...]
    T = lax.dot_general(qs, fl, (((1,), (1,)), ((), ())),
                        precision=_HI, preferred_element_type=jnp.float32)
    scores = lax.dot_general(T, oh, (((1,), (0,)), ((), ())),
                             precision=_HI, preferred_element_type=jnp.float32)
    scores = jnp.where(written, scores, -1e9)

    tops = []
    sc = scores
    for _ in range(SIMK):
        mx = jnp.max(sc, axis=1, keepdims=True)
        am = jnp.min(jnp.where(sc == mx, mm, MAXMEM), axis=1, keepdims=True)
        tops.append(am)
        sc = jnp.where(mm == am, -jnp.inf, sc)
    top = jnp.concatenate(tops, axis=1)
    base = top[:, 0:1]
    cont = jnp.concatenate([(base - 1 + MAXMEM) % MAXMEM,
                            (base + 1) % MAXMEM], axis=1)
    alli = jnp.concatenate([top, cont], axis=1)

    NSEL = SIMK + CONTK
    mm3 = jnp.reshape(mm, (1, 1, MAXMEM))
    sel = alli[:, :, None] == mm3
    st3 = jnp.reshape(startm, (1, 1, MAXMEM))
    ct3 = jnp.reshape(countm, (1, 1, MAXMEM))
    wr3 = jnp.reshape(written.astype(jnp.int32), (1, 1, MAXMEM))
    st_sel = jnp.sum(jnp.where(sel, st3, 0), axis=2)
    ct_sel = jnp.sum(jnp.where(sel, ct3, 0), axis=2)
    wr_sel = jnp.sum(jnp.where(sel, wr3, 0), axis=2)
    j = lax.broadcasted_iota(jnp.int32, (B, NSEL, MAXLEN), 2)
    tok = st_sel[:, :, None] + j
    valid = (wr_sel[:, :, None] > 0) & (j < ct_sel[:, :, None]) & (tok < NTOK)
    tokc = jnp.where(valid, jnp.clip(tok, 0, NTOK - 1), 0)
    tok2 = jnp.reshape(tokc, (B * NSEL, MAXLEN))
    val2 = jnp.reshape(valid.astype(jnp.float32), (B * NSEL, MAXLEN))
    padr = GROWS // MAXLEN - B * NSEL
    tok_ref[...] = jnp.concatenate(
        [tok2, jnp.zeros((padr, MAXLEN), jnp.int32)], axis=0)
    val_ref[...] = jnp.concatenate(
        [val2, jnp.zeros((padr, MAXLEN), jnp.float32)], axis=0)


def _plan(surp, flat, W_sim, bsim2):
    nr = GROWS // MAXLEN
    return pl.pallas_call(
        _plan_body,
        out_shape=(jax.ShapeDtypeStruct((nr, MAXLEN), jnp.int32),
                   jax.ShapeDtypeStruct((nr, MAXLEN), jnp.float32)),
    )(surp, flat, W_sim, bsim2)


@functools.cache
def _sc_gather_fn():
    @functools.partial(
        pl.kernel,
        mesh=plsc.VectorSubcoreMesh(core_axis_name="c", subcore_axis_name="s"),
        out_type=jax.ShapeDtypeStruct((GROWS, HID), jnp.float32),
        scratch_types=[
            pltpu.VMEM((RPW,), jnp.int32),
            pltpu.VMEM((RPW, HID), jnp.float32),
            pltpu.SemaphoreType.DMA,
        ],
    )
    def _sc_gather(table_hbm, idx_hbm, out_hbm, idx_v, rows_v, sem):
        wid = lax.axis_index("s") * SC_NC + lax.axis_index("c")
        base = wid * RPW
        pltpu.sync_copy(idx_hbm.at[pl.ds(base, RPW)], idx_v)
        pltpu.async_copy(table_hbm.at[idx_v], rows_v, sem).wait()
        pltpu.sync_copy(rows_v, out_hbm.at[pl.ds(base, RPW)])

    return _sc_gather


def _mlp_body(rows_ref, val_ref, w1_ref, b1_ref, w2_ref, b2_ref, out_ref):
    x = rows_ref[...] * val_ref[...]
    h = lax.dot_general(x, w1_ref[...], (((1,), (1,)), ((), ())),
                        precision=_HI, preferred_element_type=jnp.float32)
    h = jnp.maximum(h + b1_ref[...], 0.0)
    o = lax.dot_general(h, w2_ref[...], (((1,), (1,)), ((), ())),
                        precision=_HI, preferred_element_type=jnp.float32)
    out_ref[...] = o + b2_ref[...]


def _mlp(rows, val, W1, b1, W2, b2):
    return pl.pallas_call(
        _mlp_body,
        out_shape=jax.ShapeDtypeStruct((GROWS, HID), jnp.float32),
    )(rows, val, W1, b1, W2, b2)


def kernel(query, key, value, attention_mask, target_tokens, W_sim, b_sim,
           W_st1, b_st1, W_st2, b_st2, W_clu):
    b, s, d = query.shape
    flat = query.reshape(NTOK, d)
    tgt2 = target_tokens.reshape(NTOK, 1).astype(jnp.int32)

    surp = _surprise(flat, tgt2, W_clu).reshape(b, s)
    tok2, val2 = _plan(surp, flat, W_sim, b_sim.reshape(1, d))

    idx = tok2.reshape(GROWS)
    rows = _sc_gather_fn()(flat, idx)
    h = _mlp(rows, val2.reshape(GROWS, 1), W_st1, b_st1.reshape(1, d),
             W_st2, b_st2.reshape(1, d))

    nsel = SIMK + CONTK
    h = h[: b * nsel * MAXLEN].reshape(b, nsel * MAXLEN, d)
    sim_buf = h[:, : SIMK * MAXLEN]
    cont_buf = h[:, SIMK * MAXLEN:]

    key_ctx = jnp.concatenate([sim_buf, cont_buf, key], axis=1)
    val_ctx = jnp.concatenate([sim_buf, cont_buf, value], axis=1)
    n_mem = nsel * MAXLEN
    mask_ext = jnp.concatenate(
        [jnp.zeros((b, s, n_mem), attention_mask.dtype), attention_mask],
        axis=-1)
    return (query, key_ctx, val_ctx, mask_ext)


# speedup vs baseline: 2.2206x; 2.2206x over previous
import functools

import jax
import jax.numpy as jnp
from jax import lax
from jax.experimental import pallas as pl
from jax.experimental.pallas import tpu as pltpu
from jax.experimental.pallas import tpu_sc as plsc

HID = 1024
VOCAB = 32000
MAXLEN = 32
MAXMEM = 512
SIMK = 3
CONTK = 2
WIN = 32
GAMMA = 2.0
B, S = 2, 2048
NTOK = B * S
VT = 640
NVT = VOCAB // VT
RT = 1024
NRB = NTOK // RT

SC_NC, SC_NS = 2, 16
SC_NW = SC_NC * SC_NS
GROWS = 512
RPW = GROWS // SC_NW

_HI = jax.lax.Precision.HIGHEST


def _shift_lanes(x, k):
    n = x.shape[-1]
    z = jnp.zeros(x.shape[:-1] + (k,), x.dtype)
    return jnp.concatenate([z, x[..., : n - k]], axis=-1)


def _dcumsum(x):
    n = x.shape[-1]
    k = 1
    while k < n:
        x = x + _shift_lanes(x, k)
        k *= 2
    return x


def _lse_body(flat_ref, tgt_ref, w_ref, surp_ref, st_ref):
    v = pl.program_id(1)

    @pl.when(v == 0)
    def _init():
        st_ref[:, 0:1] = jnp.full((RT, 1), -1e30, jnp.float32)
        st_ref[:, 1:2] = jnp.zeros((RT, 1), jnp.float32)
        st_ref[:, 2:3] = jnp.zeros((RT, 1), jnp.float32)

    fl = flat_ref[...]
    wt = w_ref[...]
    lg = lax.dot_general(fl, wt, (((1,), (0,)), ((), ())),
                         precision=_HI, preferred_element_type=jnp.float32)
    col = v * VT + lax.broadcasted_iota(jnp.int32, (1, VT), 1)
    tgt = tgt_ref[...]
    st_ref[:, 2:3] = st_ref[:, 2:3] + jnp.sum(
        jnp.where(col == tgt, lg, 0.0), axis=1, keepdims=True)
    m_old = st_ref[:, 0:1]
    mt = jnp.max(lg, axis=1, keepdims=True)
    mn = jnp.maximum(m_old, mt)
    st_ref[:, 1:2] = st_ref[:, 1:2] * jnp.exp(m_old - mn) + jnp.sum(
        jnp.exp(lg - mn), axis=1, keepdims=True)
    st_ref[:, 0:1] = mn

    @pl.when(v == NVT - 1)
    def _fin():
        surp_ref[...] = st_ref[:, 0:1] + jnp.log(st_ref[:, 1:2]) - st_ref[:, 2:3]


def _surprise(flat, tgt2, W_clu_t):
    return pl.pallas_call(
        _lse_body,
        grid=(NRB, NVT),
        in_specs=[
            pl.BlockSpec((RT, HID), lambda r, v: (r, 0)),
            pl.BlockSpec((RT, 1), lambda r, v: (r, 0)),
            pl.BlockSpec((HID, VT), lambda r, v: (0, v)),
        ],
        out_specs=pl.BlockSpec((RT, 1), lambda r, v: (r, 0)),
        out_shape=jax.ShapeDtypeStruct((NTOK, 1), jnp.float32),
        scratch_shapes=[
            pltpu.VMEM((RT, 3), jnp.float32),
        ],
    )(flat, tgt2, W_clu_t)


def _plan_body(surp_ref, flat_ref, wsim_ref, bsim_ref, tok_ref, val_ref):
    s = surp_ref[...]
    cs = _dcumsum(s)
    cs2 = _dcumsum(s * s)
    smw = cs - _shift_lanes(cs, WIN)
    sm2w = cs2 - _shift_lanes(cs2, WIN)
    coli = lax.broadcasted_iota(jnp.int32, (1, S), 1)
    cnt = jnp.minimum(coli.astype(jnp.float32) + 1.0, float(WIN))
    mean = smw / cnt
    std = jnp.sqrt(jnp.clip(sm2w / cnt - mean * mean, 0.0, None))
    bnd = s > mean + GAMMA * std
    isnew = bnd | (coli == 0)
    inewi = isnew.astype(jnp.int32)
    R = _dcumsum(inewi)
    tot = R[:, S - 1:S]
    n_events = jnp.sum(tot)
    off = jnp.concatenate([jnp.zeros((1, 1), jnp.int32), tot[0:1]], axis=0)
    inv = R - 1 + off

    mm = lax.broadcasted_iota(jnp.int32, (1, MAXMEM), 1)
    written = mm < n_events
    qd = jnp.maximum(n_events - 1 - mm, 0) // MAXMEM
    e_m = mm + MAXMEM * qd

    countm = jnp.zeros((1, MAXMEM), jnp.int32)
    startm = jnp.zeros((1, MAXMEM), jnp.int32)
    ohs = []
    for b in range(B):
        invb = jnp.swapaxes(inv[b:b + 1, :], 0, 1)
        newb = jnp.swapaxes(inewi[b:b + 1, :], 0, 1)
        Mb = (invb == e_m) & written
        posb = b * S + lax.broadcasted_iota(jnp.int32, (S, 1), 0)
        ohs.append((Mb & (newb > 0)).astype(jnp.float32))
        countm = countm + jnp.sum(Mb.astype(jnp.int32), axis=0, keepdims=True)
        startm = startm + jnp.sum(jnp.where(Mb & (newb > 0), posb, 0),
                                  axis=0, keepdims=True)
    oh = jnp.concatenate(ohs, axis=0)

    fl = flat_ref[...]
    q0 = jnp.sum(fl[0:S], axis=0, keepdims=True) * (1.0 / S)
    q1 = jnp.sum(fl[S:NTOK], axis=0, keepdims=True) * (1.0 / S)
    qm = jnp.concatenate([q0, q1], axis=0)
    qs = lax.dot_general(qm, wsim_ref[...], (((1,), (1,)), ((), ())),
                         precision=_HI, preferred_element_type=jnp.float32)
    qs = qs + bsim_ref[...]
    T = lax.dot_general(qs, fl, (((1,), (1,)), ((), ())),
                        precision=_HI, preferred_element_type=jnp.float32)
    scores = lax.dot_general(T, oh, (((1,), (0,)), ((), ())),
                             precision=_HI, preferred_element_type=jnp.float32)
    scores = jnp.where(written, scores, -1e9)

    tops = []
    sc = scores
    for _ in range(SIMK):
        mx = jnp.max(sc, axis=1, keepdims=True)
        am = jnp.min(jnp.where(sc == mx, mm, MAXMEM), axis=1, keepdims=True)
        tops.append(am)
        sc = jnp.where(mm == am, -jnp.inf, sc)
    top = jnp.concatenate(tops, axis=1)
    base = top[:, 0:1]
    cont = jnp.concatenate([(base - 1 + MAXMEM) % MAXMEM,
                            (base + 1) % MAXMEM], axis=1)
    alli = jnp.concatenate([top, cont], axis=1)

    NSEL = SIMK + CONTK
    mm3 = jnp.reshape(mm, (1, 1, MAXMEM))
    sel = alli[:, :, None] == mm3
    st3 = jnp.reshape(startm, (1, 1, MAXMEM))
    ct3 = jnp.reshape(countm, (1, 1, MAXMEM))
    wr3 = jnp.reshape(written.astype(jnp.int32), (1, 1, MAXMEM))
    st_sel = jnp.sum(jnp.where(sel, st3, 0), axis=2)
    ct_sel = jnp.sum(jnp.where(sel, ct3, 0), axis=2)
    wr_sel = jnp.sum(jnp.where(sel, wr3, 0), axis=2)
    j = lax.broadcasted_iota(jnp.int32, (B, NSEL, MAXLEN), 2)
    tok = st_sel[:, :, None] + j
    valid = (wr_sel[:, :, None] > 0) & (j < ct_sel[:, :, None]) & (tok < NTOK)
    tokc = jnp.where(valid, jnp.clip(tok, 0, NTOK - 1), 0)
    tok2 = jnp.reshape(tokc, (B * NSEL, MAXLEN))
    val2 = jnp.reshape(valid.astype(jnp.float32), (B * NSEL, MAXLEN))
    padr = GROWS // MAXLEN - B * NSEL
    tok_ref[...] = jnp.concatenate(
        [tok2, jnp.zeros((padr, MAXLEN), jnp.int32)], axis=0)
    val_ref[...] = jnp.concatenate(
        [val2, jnp.zeros((padr, MAXLEN), jnp.float32)], axis=0)


def _plan(surp, flat, W_sim, bsim2):
    nr = GROWS // MAXLEN
    return pl.pallas_call(
        _plan_body,
        out_shape=(jax.ShapeDtypeStruct((nr, MAXLEN), jnp.int32),
                   jax.ShapeDtypeStruct((nr, MAXLEN), jnp.float32)),
    )(surp, flat, W_sim, bsim2)


@functools.cache
def _sc_gather_fn():
    @functools.partial(
        pl.kernel,
        mesh=plsc.VectorSubcoreMesh(core_axis_name="c", subcore_axis_name="s"),
        out_type=jax.ShapeDtypeStruct((GROWS, HID), jnp.float32),
        scratch_types=[
            pltpu.VMEM((RPW,), jnp.int32),
            pltpu.VMEM((RPW, HID), jnp.float32),
            pltpu.SemaphoreType.DMA,
        ],
    )
    def _sc_gather(table_hbm, idx_hbm, out_hbm, idx_v, rows_v, sem):
        wid = lax.axis_index("s") * SC_NC + lax.axis_index("c")
        base = wid * RPW
        pltpu.sync_copy(idx_hbm.at[pl.ds(base, RPW)], idx_v)
        pltpu.async_copy(table_hbm.at[idx_v], rows_v, sem).wait()
        pltpu.sync_copy(rows_v, out_hbm.at[pl.ds(base, RPW)])

    return _sc_gather


def _mlp_body(rows_ref, val_ref, w1_ref, b1_ref, w2_ref, b2_ref, out_ref):
    x = rows_ref[...] * val_ref[...]
    h = lax.dot_general(x, w1_ref[...], (((1,), (1,)), ((), ())),
                        precision=_HI, preferred_element_type=jnp.float32)
    h = jnp.maximum(h + b1_ref[...], 0.0)
    o = lax.dot_general(h, w2_ref[...], (((1,), (1,)), ((), ())),
                        precision=_HI, preferred_element_type=jnp.float32)
    out_ref[...] = o + b2_ref[...]


def _mlp(rows, val, W1, b1, W2, b2):
    return pl.pallas_call(
        _mlp_body,
        out_shape=jax.ShapeDtypeStruct((GROWS, HID), jnp.float32),
    )(rows, val, W1, b1, W2, b2)


def kernel(query, key, value, attention_mask, target_tokens, W_sim, b_sim,
           W_st1, b_st1, W_st2, b_st2, W_clu):
    b, s, d = query.shape
    flat = query.reshape(NTOK, d)
    tgt2 = target_tokens.reshape(NTOK, 1).astype(jnp.int32)

    surp = _surprise(flat, tgt2, W_clu.T).reshape(b, s)
    tok2, val2 = _plan(surp, flat, W_sim, b_sim.reshape(1, d))

    idx = tok2.reshape(GROWS)
    rows = _sc_gather_fn()(flat, idx)
    h = _mlp(rows, val2.reshape(GROWS, 1), W_st1, b_st1.reshape(1, d),
             W_st2, b_st2.reshape(1, d))

    nsel = SIMK + CONTK
    h = h[: b * nsel * MAXLEN].reshape(b, nsel * MAXLEN, d)
    sim_buf = h[:, : SIMK * MAXLEN]
    cont_buf = h[:, SIMK * MAXLEN:]

    key_ctx = jnp.concatenate([sim_buf, cont_buf, key], axis=1)
    val_ctx = jnp.concatenate([sim_buf, cont_buf, value], axis=1)
    n_mem = nsel * MAXLEN
    mask_ext = jnp.concatenate(
        [jnp.zeros((b, s, n_mem), attention_mask.dtype), attention_mask],
        axis=-1)
    return (query, key_ctx, val_ctx, mask_ext)


# speedup vs baseline: 2.3749x; 1.0695x over previous
import functools

import jax
import jax.numpy as jnp
from jax import lax
from jax.experimental import pallas as pl
from jax.experimental.pallas import tpu as pltpu
from jax.experimental.pallas import tpu_sc as plsc

HID = 1024
VOCAB = 32000
MAXLEN = 32
MAXMEM = 512
SIMK = 3
CONTK = 2
WIN = 32
GAMMA = 2.0
B, S = 2, 2048
NTOK = B * S
VT = 640
NVT = VOCAB // VT
RT = 1024
NRB = NTOK // RT

SC_NC, SC_NS = 2, 16
SC_NW = SC_NC * SC_NS
GROWS = 512
RPW = GROWS // SC_NW

_HI = jax.lax.Precision.HIGHEST


def _shift_lanes(x, k):
    n = x.shape[-1]
    z = jnp.zeros(x.shape[:-1] + (k,), x.dtype)
    return jnp.concatenate([z, x[..., : n - k]], axis=-1)


def _dcumsum(x):
    n = x.shape[-1]
    k = 1
    while k < n:
        x = x + _shift_lanes(x, k)
        k *= 2
    return x


def _lse_body(flat_ref, tgt_ref, w_ref, surp_ref, st_ref, lgs_ref):
    v = pl.program_id(1)

    @pl.when(v == 0)
    def _init():
        st_ref[:, 0:1] = jnp.full((RT, 1), -1e30, jnp.float32)
        st_ref[:, 1:2] = jnp.zeros((RT, 1), jnp.float32)
        st_ref[:, 2:3] = jnp.zeros((RT, 1), jnp.float32)

    @pl.when(v > 0)
    def _stats():
        lg = lgs_ref[...]
        col = (v - 1) * VT + lax.broadcasted_iota(jnp.int32, (1, VT), 1)
        tgt = tgt_ref[...]
        st_ref[:, 2:3] = st_ref[:, 2:3] + jnp.sum(
            jnp.where(col == tgt, lg, 0.0), axis=1, keepdims=True)
        m_old = st_ref[:, 0:1]
        mt = jnp.max(lg, axis=1, keepdims=True)
        mn = jnp.maximum(m_old, mt)
        st_ref[:, 1:2] = st_ref[:, 1:2] * jnp.exp(m_old - mn) + jnp.sum(
            jnp.exp(lg - mn), axis=1, keepdims=True)
        st_ref[:, 0:1] = mn

    @pl.when(v < NVT)
    def _mm():
        lgs_ref[...] = lax.dot_general(
            flat_ref[...], w_ref[...], (((1,), (0,)), ((), ())),
            precision=_HI, preferred_element_type=jnp.float32)

    @pl.when(v == NVT)
    def _fin():
        surp_ref[...] = st_ref[:, 0:1] + jnp.log(st_ref[:, 1:2]) - st_ref[:, 2:3]


def _surprise(flat, tgt2, W_clu_t):
    return pl.pallas_call(
        _lse_body,
        grid=(NRB, NVT + 1),
        in_specs=[
            pl.BlockSpec((RT, HID), lambda r, v: (r, 0)),
            pl.BlockSpec((RT, 1), lambda r, v: (r, 0)),
            pl.BlockSpec((HID, VT), lambda r, v: (0, jnp.minimum(v, NVT - 1))),
        ],
        out_specs=pl.BlockSpec((RT, 1), lambda r, v: (r, 0)),
        out_shape=jax.ShapeDtypeStruct((NTOK, 1), jnp.float32),
        scratch_shapes=[
            pltpu.VMEM((RT, 3), jnp.float32),
            pltpu.VMEM((RT, VT), jnp.float32),
        ],
        compiler_params=pltpu.CompilerParams(
            dimension_semantics=("parallel", "arbitrary")),
    )(flat, tgt2, W_clu_t)


def _plan_body(surp_ref, flat_ref, wsim_ref, bsim_ref, tok_ref, val_ref):
    s = surp_ref[...]
    cs = _dcumsum(s)
    cs2 = _dcumsum(s * s)
    smw = cs - _shift_lanes(cs, WIN)
    sm2w = cs2 - _shift_lanes(cs2, WIN)
    coli = lax.broadcasted_iota(jnp.int32, (1, S), 1)
    cnt = jnp.minimum(coli.astype(jnp.float32) + 1.0, float(WIN))
    mean = smw / cnt
    std = jnp.sqrt(jnp.clip(sm2w / cnt - mean * mean, 0.0, None))
    bnd = s > mean + GAMMA * std
    isnew = bnd | (coli == 0)
    inewi = isnew.astype(jnp.int32)
    R = _dcumsum(inewi)
    tot = R[:, S - 1:S]
    n_events = jnp.sum(tot)
    off = jnp.concatenate([jnp.zeros((1, 1), jnp.int32), tot[0:1]], axis=0)
    inv = R - 1 + off

    mm = lax.broadcasted_iota(jnp.int32, (1, MAXMEM), 1)
    written = mm < n_events
    qd = jnp.maximum(n_events - 1 - mm, 0) // MAXMEM
    e_m = mm + MAXMEM * qd

    countm = jnp.zeros((1, MAXMEM), jnp.int32)
    startm = jnp.zeros((1, MAXMEM), jnp.int32)
    ohs = []
    for b in range(B):
        invb = jnp.swapaxes(inv[b:b + 1, :], 0, 1)
        newb = jnp.swapaxes(inewi[b:b + 1, :], 0, 1)
        Mb = (invb == e_m) & written
        posb = b * S + lax.broadcasted_iota(jnp.int32, (S, 1), 0)
        ohs.append((Mb & (newb > 0)).astype(jnp.float32))
        countm = countm + jnp.sum(Mb.astype(jnp.int32), axis=0, keepdims=True)
        startm = startm + jnp.sum(jnp.where(Mb & (newb > 0), posb, 0),
                                  axis=0, keepdims=True)
    oh = jnp.concatenate(ohs, axis=0)

    fl = flat_ref[...]
    q0 = jnp.sum(fl[0:S], axis=0, keepdims=True) * (1.0 / S)
    q1 = jnp.sum(fl[S:NTOK], axis=0, keepdims=True) * (1.0 / S)
    qm = jnp.concatenate([q0, q1], axis=0)
    qs = lax.dot_general(qm, wsim_ref[...], (((1,), (1,)), ((), ())),
                         precision=_HI, preferred_element_type=jnp.float32)
    qs = qs + bsim_ref[...]
    T = lax.dot_general(qs, fl, (((1,), (1,)), ((), ())),
                        precision=_HI, preferred_element_type=jnp.float32)
    scores = lax.dot_general(T, oh, (((1,), (0,)), ((), ())),
                             precision=_HI, preferred_element_type=jnp.float32)
    scores = jnp.where(written, scores, -1e9)

    tops = []
    sc = scores
    for _ in range(SIMK):
        mx = jnp.max(sc, axis=1, keepdims=True)
        am = jnp.min(jnp.where(sc == mx, mm, MAXMEM), axis=1, keepdims=True)
        tops.append(am)
        sc = jnp.where(mm == am, -jnp.inf, sc)
    top = jnp.concatenate(tops, axis=1)
    base = top[:, 0:1]
    cont = jnp.concatenate([(base - 1 + MAXMEM) % MAXMEM,
                            (base + 1) % MAXMEM], axis=1)
    alli = jnp.concatenate([top, cont], axis=1)

    NSEL = SIMK + CONTK
    mm3 = jnp.reshape(mm, (1, 1, MAXMEM))
    sel = alli[:, :, None] == mm3
    st3 = jnp.reshape(startm, (1, 1, MAXMEM))
    ct3 = jnp.reshape(countm, (1, 1, MAXMEM))
    wr3 = jnp.reshape(written.astype(jnp.int32), (1, 1, MAXMEM))
    st_sel = jnp.sum(jnp.where(sel, st3, 0), axis=2)
    ct_sel = jnp.sum(jnp.where(sel, ct3, 0), axis=2)
    wr_sel = jnp.sum(jnp.where(sel, wr3, 0), axis=2)
    j = lax.broadcasted_iota(jnp.int32, (B, NSEL, MAXLEN), 2)
    tok = st_sel[:, :, None] + j
    valid = (wr_sel[:, :, None] > 0) & (j < ct_sel[:, :, None]) & (tok < NTOK)
    tokc = jnp.where(valid, jnp.clip(tok, 0, NTOK - 1), 0)
    tok2 = jnp.reshape(tokc, (B * NSEL, MAXLEN))
    val2 = jnp.reshape(valid.astype(jnp.float32), (B * NSEL, MAXLEN))
    padr = GROWS // MAXLEN - B * NSEL
    tok_ref[...] = jnp.concatenate(
        [tok2, jnp.zeros((padr, MAXLEN), jnp.int32)], axis=0)
    val_ref[...] = jnp.concatenate(
        [val2, jnp.zeros((padr, MAXLEN), jnp.float32)], axis=0)


def _plan(surp, flat, W_sim, bsim2):
    nr = GROWS // MAXLEN
    return pl.pallas_call(
        _plan_body,
        out_shape=(jax.ShapeDtypeStruct((nr, MAXLEN), jnp.int32),
                   jax.ShapeDtypeStruct((nr, MAXLEN), jnp.float32)),
    )(surp, flat, W_sim, bsim2)


@functools.cache
def _sc_gather_fn():
    @functools.partial(
        pl.kernel,
        mesh=plsc.VectorSubcoreMesh(core_axis_name="c", subcore_axis_name="s"),
        out_type=jax.ShapeDtypeStruct((GROWS, HID), jnp.float32),
        scratch_types=[
            pltpu.VMEM((RPW,), jnp.int32),
            pltpu.VMEM((RPW, HID), jnp.float32),
            pltpu.SemaphoreType.DMA,
        ],
    )
    def _sc_gather(table_hbm, idx_hbm, out_hbm, idx_v, rows_v, sem):
        wid = lax.axis_index("s") * SC_NC + lax.axis_index("c")
        base = wid * RPW
        pltpu.sync_copy(idx_hbm.at[pl.ds(base, RPW)], idx_v)
        pltpu.async_copy(table_hbm.at[idx_v], rows_v, sem).wait()
        pltpu.sync_copy(rows_v, out_hbm.at[pl.ds(base, RPW)])

    return _sc_gather


def _mlp_body(rows_ref, val_ref, w1_ref, b1_ref, w2_ref, b2_ref, out_ref):
    x = rows_ref[...] * val_ref[...]
    h = lax.dot_general(x, w1_ref[...], (((1,), (1,)), ((), ())),
                        precision=_HI, preferred_element_type=jnp.float32)
    h = jnp.maximum(h + b1_ref[...], 0.0)
    o = lax.dot_general(h, w2_ref[...], (((1,), (1,)), ((), ())),
                        precision=_HI, preferred_element_type=jnp.float32)
    out_ref[...] = o + b2_ref[...]


def _mlp(rows, val, W1, b1, W2, b2):
    return pl.pallas_call(
        _mlp_body,
        out_shape=jax.ShapeDtypeStruct((GROWS, HID), jnp.float32),
    )(rows, val, W1, b1, W2, b2)


def kernel(query, key, value, attention_mask, target_tokens, W_sim, b_sim,
           W_st1, b_st1, W_st2, b_st2, W_clu):
    b, s, d = query.shape
    flat = query.reshape(NTOK, d)
    tgt2 = target_tokens.reshape(NTOK, 1).astype(jnp.int32)

    surp = _surprise(flat, tgt2, W_clu.T).reshape(b, s)
    tok2, val2 = _plan(surp, flat, W_sim, b_sim.reshape(1, d))

    idx = tok2.reshape(GROWS)
    rows = _sc_gather_fn()(flat, idx)
    h = _mlp(rows, val2.reshape(GROWS, 1), W_st1, b_st1.reshape(1, d),
             W_st2, b_st2.reshape(1, d))

    nsel = SIMK + CONTK
    h = h[: b * nsel * MAXLEN].reshape(b, nsel * MAXLEN, d)
    sim_buf = h[:, : SIMK * MAXLEN]
    cont_buf = h[:, SIMK * MAXLEN:]

    key_ctx = jnp.concatenate([sim_buf, cont_buf, key], axis=1)
    val_ctx = jnp.concatenate([sim_buf, cont_buf, value], axis=1)
    n_mem = nsel * MAXLEN
    mask_ext = jnp.concatenate(
        [jnp.zeros((b, s, n_mem), attention_mask.dtype), attention_mask],
        axis=-1)
    return (query, key_ctx, val_ctx, mask_ext)


# speedup vs baseline: 7.0410x; 2.9647x over previous
import functools

import jax
import jax.numpy as jnp
from jax import lax
from jax.experimental import pallas as pl
from jax.experimental.pallas import tpu as pltpu
from jax.experimental.pallas import tpu_sc as plsc

HID = 1024
VOCAB = 32000
MAXLEN = 32
MAXMEM = 512
SIMK = 3
CONTK = 2
WIN = 32
GAMMA = 2.0
B, S = 2, 2048
NTOK = B * S
VT = 640
NVT = VOCAB // VT
RT = 1024
NRB = NTOK // RT

SC_NC, SC_NS = 2, 16
SC_NW = SC_NC * SC_NS
GROWS = 512
RPW = GROWS // SC_NW

_HI = jax.lax.Precision.HIGHEST


def _shift_lanes(x, k):
    n = x.shape[-1]
    z = jnp.zeros(x.shape[:-1] + (k,), x.dtype)
    return jnp.concatenate([z, x[..., : n - k]], axis=-1)


def _dcumsum(x):
    n = x.shape[-1]
    k = 1
    while k < n:
        x = x + _shift_lanes(x, k)
        k *= 2
    return x


def _lse_body(flat_ref, tgt_ref, w_ref, surp_ref, st_ref, lgs_ref):
    v = pl.program_id(1)

    @pl.when(v == 0)
    def _init():
        st_ref[:, 0:1] = jnp.full((RT, 1), -1e30, jnp.float32)
        st_ref[:, 1:2] = jnp.zeros((RT, 1), jnp.float32)
        st_ref[:, 2:3] = jnp.zeros((RT, 1), jnp.float32)

    @pl.when(v > 0)
    def _stats():
        lg = lgs_ref[...]
        col = (v - 1) * VT + lax.broadcasted_iota(jnp.int32, (1, VT), 1)
        tgt = tgt_ref[...]
        st_ref[:, 2:3] = st_ref[:, 2:3] + jnp.sum(
            jnp.where(col == tgt, lg, 0.0), axis=1, keepdims=True)
        m_old = st_ref[:, 0:1]
        mt = jnp.max(lg, axis=1, keepdims=True)
        mn = jnp.maximum(m_old, mt)
        st_ref[:, 1:2] = st_ref[:, 1:2] * jnp.exp(m_old - mn) + jnp.sum(
            jnp.exp(lg - mn), axis=1, keepdims=True)
        st_ref[:, 0:1] = mn

    @pl.when(v < NVT)
    def _mm():
        lgs_ref[...] = lax.dot_general(
            flat_ref[...], w_ref[...], (((1,), (0,)), ((), ())),
            preferred_element_type=jnp.float32)

    @pl.when(v == NVT)
    def _fin():
        surp_ref[...] = st_ref[:, 0:1] + jnp.log(st_ref[:, 1:2]) - st_ref[:, 2:3]


def _surprise(flat, tgt2, W_clu_t):
    return pl.pallas_call(
        _lse_body,
        grid=(NRB, NVT + 1),
        in_specs=[
            pl.BlockSpec((RT, HID), lambda r, v: (r, 0)),
            pl.BlockSpec((RT, 1), lambda r, v: (r, 0)),
            pl.BlockSpec((HID, VT), lambda r, v: (0, jnp.minimum(v, NVT - 1))),
        ],
        out_specs=pl.BlockSpec((RT, 1), lambda r, v: (r, 0)),
        out_shape=jax.ShapeDtypeStruct((NTOK, 1), jnp.float32),
        scratch_shapes=[
            pltpu.VMEM((RT, 3), jnp.float32),
            pltpu.VMEM((RT, VT), jnp.float32),
        ],
        compiler_params=pltpu.CompilerParams(
            dimension_semantics=("parallel", "arbitrary")),
    )(flat, tgt2, W_clu_t)


def _plan_body(surp_ref, flat_ref, wsim_ref, bsim_ref, tok_ref, val_ref):
    s = surp_ref[...]
    cs = _dcumsum(s)
    cs2 = _dcumsum(s * s)
    smw = cs - _shift_lanes(cs, WIN)
    sm2w = cs2 - _shift_lanes(cs2, WIN)
    coli = lax.broadcasted_iota(jnp.int32, (1, S), 1)
    cnt = jnp.minimum(coli.astype(jnp.float32) + 1.0, float(WIN))
    mean = smw / cnt
    std = jnp.sqrt(jnp.clip(sm2w / cnt - mean * mean, 0.0, None))
    bnd = s > mean + GAMMA * std
    isnew = bnd | (coli == 0)
    inewi = isnew.astype(jnp.int32)
    R = _dcumsum(inewi)
    tot = R[:, S - 1:S]
    n_events = jnp.sum(tot)
    off = jnp.concatenate([jnp.zeros((1, 1), jnp.int32), tot[0:1]], axis=0)
    inv = R - 1 + off

    mm = lax.broadcasted_iota(jnp.int32, (1, MAXMEM), 1)
    written = mm < n_events
    qd = jnp.maximum(n_events - 1 - mm, 0) // MAXMEM
    e_m = mm + MAXMEM * qd

    countm = jnp.zeros((1, MAXMEM), jnp.int32)
    startm = jnp.zeros((1, MAXMEM), jnp.int32)
    ohs = []
    for b in range(B):
        invb = jnp.swapaxes(inv[b:b + 1, :], 0, 1)
        newb = jnp.swapaxes(inewi[b:b + 1, :], 0, 1)
        Mb = (invb == e_m) & written
        posb = b * S + lax.broadcasted_iota(jnp.int32, (S, 1), 0)
        ohs.append((Mb & (newb > 0)).astype(jnp.float32))
        countm = countm + jnp.sum(Mb.astype(jnp.int32), axis=0, keepdims=True)
        startm = startm + jnp.sum(jnp.where(Mb & (newb > 0), posb, 0),
                                  axis=0, keepdims=True)
    oh = jnp.concatenate(ohs, axis=0)

    fl = flat_ref[...]
    q0 = jnp.sum(fl[0:S], axis=0, keepdims=True) * (1.0 / S)
    q1 = jnp.sum(fl[S:NTOK], axis=0, keepdims=True) * (1.0 / S)
    qm = jnp.concatenate([q0, q1], axis=0)
    qs = lax.dot_general(qm, wsim_ref[...], (((1,), (1,)), ((), ())),
                         precision=_HI, preferred_element_type=jnp.float32)
    qs = qs + bsim_ref[...]
    T = lax.dot_general(qs, fl, (((1,), (1,)), ((), ())),
                        precision=_HI, preferred_element_type=jnp.float32)
    scores = lax.dot_general(T, oh, (((1,), (0,)), ((), ())),
                             precision=_HI, preferred_element_type=jnp.float32)
    scores = jnp.where(written, scores, -1e9)

    tops = []
    sc = scores
    for _ in range(SIMK):
        mx = jnp.max(sc, axis=1, keepdims=True)
        am = jnp.min(jnp.where(sc == mx, mm, MAXMEM), axis=1, keepdims=True)
        tops.append(am)
        sc = jnp.where(mm == am, -jnp.inf, sc)
    top = jnp.concatenate(tops, axis=1)
    base = top[:, 0:1]
    cont = jnp.concatenate([(base - 1 + MAXMEM) % MAXMEM,
                            (base + 1) % MAXMEM], axis=1)
    alli = jnp.concatenate([top, cont], axis=1)

    NSEL = SIMK + CONTK
    mm3 = jnp.reshape(mm, (1, 1, MAXMEM))
    sel = alli[:, :, None] == mm3
    st3 = jnp.reshape(startm, (1, 1, MAXMEM))
    ct3 = jnp.reshape(countm, (1, 1, MAXMEM))
    wr3 = jnp.reshape(written.astype(jnp.int32), (1, 1, MAXMEM))
    st_sel = jnp.sum(jnp.where(sel, st3, 0), axis=2)
    ct_sel = jnp.sum(jnp.where(sel, ct3, 0), axis=2)
    wr_sel = jnp.sum(jnp.where(sel, wr3, 0), axis=2)
    j = lax.broadcasted_iota(jnp.int32, (B, NSEL, MAXLEN), 2)
    tok = st_sel[:, :, None] + j
    valid = (wr_sel[:, :, None] > 0) & (j < ct_sel[:, :, None]) & (tok < NTOK)
    tokc = jnp.where(valid, jnp.clip(tok, 0, NTOK - 1), 0)
    tok2 = jnp.reshape(tokc, (B * NSEL, MAXLEN))
    val2 = jnp.reshape(valid.astype(jnp.float32), (B * NSEL, MAXLEN))
    padr = GROWS // MAXLEN - B * NSEL
    tok_ref[...] = jnp.concatenate(
        [tok2, jnp.zeros((padr, MAXLEN), jnp.int32)], axis=0)
    val_ref[...] = jnp.concatenate(
        [val2, jnp.zeros((padr, MAXLEN), jnp.float32)], axis=0)


def _plan(surp, flat, W_sim, bsim2):
    nr = GROWS // MAXLEN
    return pl.pallas_call(
        _plan_body,
        out_shape=(jax.ShapeDtypeStruct((nr, MAXLEN), jnp.int32),
                   jax.ShapeDtypeStruct((nr, MAXLEN), jnp.float32)),
    )(surp, flat, W_sim, bsim2)


@functools.cache
def _sc_gather_fn():
    @functools.partial(
        pl.kernel,
        mesh=plsc.VectorSubcoreMesh(core_axis_name="c", subcore_axis_name="s"),
        out_type=jax.ShapeDtypeStruct((GROWS, HID), jnp.float32),
        scratch_types=[
            pltpu.VMEM((RPW,), jnp.int32),
            pltpu.VMEM((RPW, HID), jnp.float32),
            pltpu.SemaphoreType.DMA,
        ],
    )
    def _sc_gather(table_hbm, idx_hbm, out_hbm, idx_v, rows_v, sem):
        wid = lax.axis_index("s") * SC_NC + lax.axis_index("c")
        base = wid * RPW
        pltpu.sync_copy(idx_hbm.at[pl.ds(base, RPW)], idx_v)
        pltpu.async_copy(table_hbm.at[idx_v], rows_v, sem).wait()
        pltpu.sync_copy(rows_v, out_hbm.at[pl.ds(base, RPW)])

    return _sc_gather


def _mlp_body(rows_ref, val_ref, w1_ref, b1_ref, w2_ref, b2_ref, out_ref):
    x = rows_ref[...] * val_ref[...]
    h = lax.dot_general(x, w1_ref[...], (((1,), (1,)), ((), ())),
                        precision=_HI, preferred_element_type=jnp.float32)
    h = jnp.maximum(h + b1_ref[...], 0.0)
    o = lax.dot_general(h, w2_ref[...], (((1,), (1,)), ((), ())),
                        precision=_HI, preferred_element_type=jnp.float32)
    out_ref[...] = o + b2_ref[...]


def _mlp(rows, val, W1, b1, W2, b2):
    return pl.pallas_call(
        _mlp_body,
        out_shape=jax.ShapeDtypeStruct((GROWS, HID), jnp.float32),
    )(rows, val, W1, b1, W2, b2)


def kernel(query, key, value, attention_mask, target_tokens, W_sim, b_sim,
           W_st1, b_st1, W_st2, b_st2, W_clu):
    b, s, d = query.shape
    flat = query.reshape(NTOK, d)
    tgt2 = target_tokens.reshape(NTOK, 1).astype(jnp.int32)

    surp = _surprise(flat, tgt2, W_clu.T).reshape(b, s)
    tok2, val2 = _plan(surp, flat, W_sim, b_sim.reshape(1, d))

    idx = tok2.reshape(GROWS)
    rows = _sc_gather_fn()(flat, idx)
    h = _mlp(rows, val2.reshape(GROWS, 1), W_st1, b_st1.reshape(1, d),
             W_st2, b_st2.reshape(1, d))

    nsel = SIMK + CONTK
    h = h[: b * nsel * MAXLEN].reshape(b, nsel * MAXLEN, d)
    sim_buf = h[:, : SIMK * MAXLEN]
    cont_buf = h[:, SIMK * MAXLEN:]

    key_ctx = jnp.concatenate([sim_buf, cont_buf, key], axis=1)
    val_ctx = jnp.concatenate([sim_buf, cont_buf, value], axis=1)
    n_mem = nsel * MAXLEN
    mask_ext = jnp.concatenate(
        [jnp.zeros((b, s, n_mem), attention_mask.dtype), attention_mask],
        axis=-1)
    return (query, key_ctx, val_ctx, mask_ext)
